# Initial kernel scaffold; baseline (speedup 1.0000x reference)
import jax
import jax.numpy as jnp
from jax.experimental import pallas as pl


def kernel(inputs, pe):
    raise NotImplementedError("write your pallas kernel here")


# speedup vs baseline: 1.6695x; 1.6695x over previous
import jax
import jax.numpy as jnp
from jax.experimental import pallas as pl


def _add_body(x_ref, pe_ref, o_ref):
    o_ref[...] = x_ref[...] + pe_ref[...][None, :, :]


def kernel(inputs, pe):
    B, S, H = inputs.shape
    BS = 512
    grid = (S // BS, B)
    return pl.pallas_call(
        _add_body,
        grid=grid,
        in_specs=[
            pl.BlockSpec((1, BS, H), lambda i, b: (b, i, 0)),
            pl.BlockSpec((BS, H), lambda i, b: (i, 0)),
        ],
        out_specs=pl.BlockSpec((1, BS, H), lambda i, b: (b, i, 0)),
        out_shape=jax.ShapeDtypeStruct((B, S, H), inputs.dtype),
    )(inputs, pe)
